# baseline (speedup 1.0000x reference)
import jax
import jax.numpy as jnp
from jax import lax
from jax.experimental import pallas as pl
from jax.experimental.pallas import tpu as pltpu
from jax.experimental.pallas import tpu_sc as plsc

N = 10000
E = 320000
NC = 2
NS = 16
NW = NC * NS
CH = 80
ECH = E // CH
RPW = ECH // NW
RPS = ECH // NS
NPAD = 10240
RPT = NPAD // NS
DH = 128
ROW_BLK = 1000
LEAK = 0.2


def _bfi(t):
    n, d = t.shape
    return (t.reshape(n, d // 32, 2, 16).swapaxes(-1, -2)
            .reshape(n, d).astype(jnp.bfloat16))


def _mesh():
    return plsc.VectorSubcoreMesh(
        core_axis_name="c", subcore_axis_name="s", num_cores=NC, num_subcores=NS
    )


_SC_PARAMS = pltpu.CompilerParams(
    needs_layout_passes=False, use_tc_tiling_on_sc=False)


def _mm3_body(x_ref, wl_ref, wr_ref, wres_ref, b_ref, xl_ref, xr_ref, res_ref):
    x = x_ref[...]
    xl_ref[...] = jnp.dot(x, wl_ref[...], preferred_element_type=jnp.float32)
    xr_ref[...] = jnp.dot(x, wr_ref[...], preferred_element_type=jnp.float32)
    res_ref[...] = (
        jnp.dot(x, wres_ref[...], preferred_element_type=jnp.float32) + b_ref[...]
    )


def _mm3(x, Wl, Wr, Wres, b):
    n, d_in = x.shape
    d_out = Wl.shape[1]
    out_sd = jax.ShapeDtypeStruct((n, d_out), jnp.float32)
    w_spec = pl.BlockSpec((d_in, d_out), lambda i: (0, 0))
    b_spec = pl.BlockSpec((1, d_out), lambda i: (0, 0))
    row_spec = pl.BlockSpec((ROW_BLK, d_in), lambda i: (i, 0))
    out_spec = pl.BlockSpec((ROW_BLK, d_out), lambda i: (i, 0))
    return pl.pallas_call(
        _mm3_body,
        grid=(n // ROW_BLK,),
        in_specs=[row_spec, w_spec, w_spec, w_spec, b_spec],
        out_specs=[out_spec, out_spec, out_spec],
        out_shape=[out_sd, out_sd, out_sd],
    )(x, Wl, Wr, Wres, b.reshape(1, d_out))


def _combine_mm_body(part_ref, res1_ref, *refs):
    (wla, wlb, wra, wrb, wsa, wsb, ba, bb,
     xla_ref, xlb_ref, xra_ref, xrb_ref, resa_ref, resb_ref) = refs
    h1 = jnp.maximum(part_ref[0] + part_ref[1] + res1_ref[...], 0.0)
    dot = lambda w: jnp.dot(h1, w[...], preferred_element_type=jnp.float32)
    xla_ref[...] = dot(wla)
    xlb_ref[...] = dot(wlb)
    xra_ref[...] = dot(wra)
    xrb_ref[...] = dot(wrb)
    resa_ref[...] = dot(wsa) + ba[...]
    resb_ref[...] = dot(wsb) + bb[...]


def _combine_mm(part1, res1, Wl2, Wr2, Wres2, b2):
    out_sd = jax.ShapeDtypeStruct((N, DH), jnp.float32)
    part_spec = pl.BlockSpec((NC, ROW_BLK, DH), lambda i: (0, i, 0))
    row_spec = pl.BlockSpec((ROW_BLK, DH), lambda i: (i, 0))
    w_spec = pl.BlockSpec((DH, DH), lambda i: (0, 0))
    b_spec = pl.BlockSpec((1, DH), lambda i: (0, 0))
    ws = [Wl2[:, :DH], Wl2[:, DH:], Wr2[:, :DH], Wr2[:, DH:],
          Wres2[:, :DH], Wres2[:, DH:]]
    bs = [b2[:DH].reshape(1, DH), b2[DH:].reshape(1, DH)]
    return pl.pallas_call(
        _combine_mm_body,
        grid=(N // ROW_BLK,),
        in_specs=[part_spec, row_spec] + [w_spec] * 6 + [b_spec] * 2,
        out_specs=[row_spec] * 6,
        out_shape=[out_sd] * 6,
    )(part1[:, :N], res1, *ws, *bs)


def _final_body(pa_ref, pb_ref, resa_ref, resb_ref, out_ref):
    out_ref[:, :DH] = pa_ref[0] + pa_ref[1] + resa_ref[...]
    out_ref[:, DH:] = pb_ref[0] + pb_ref[1] + resb_ref[...]


def _final(partA, partB, res2a, res2b):
    part_spec = pl.BlockSpec((NC, ROW_BLK, DH), lambda i: (0, i, 0))
    row_spec = pl.BlockSpec((ROW_BLK, DH), lambda i: (i, 0))
    return pl.pallas_call(
        _final_body,
        grid=(N // ROW_BLK,),
        in_specs=[part_spec, part_spec, row_spec, row_spec],
        out_specs=pl.BlockSpec((ROW_BLK, 2 * DH), lambda i: (i, 0)),
        out_shape=jax.ShapeDtypeStruct((N, 2 * DH), jnp.float32),
    )(partA[:, :N], partB[:, :N], res2a, res2b)


def _edge_logits(src2, dst2, xls, xrs, a02, a08):
    nh = len(xls)

    def body(*refs):
        it = iter(refs)
        src_hbm = next(it); dst_hbm = next(it)
        xl_hbm = [next(it) for _ in range(nh)]
        xr_hbm = [next(it) for _ in range(nh)]
        a02_hbm = next(it); a08_hbm = next(it)
        e_hbm = next(it); emax_hbm = next(it)
        srcb = next(it); dstb = next(it); eb = next(it)
        gl = [[next(it) for _ in range(nh)] for _ in range(2)]
        gr = [[next(it) for _ in range(nh)] for _ in range(2)]
        a02_v = next(it); a08_v = next(it); accm = next(it); mx_v = next(it)
        sems = [next(it) for _ in range(2)]

        wid = lax.axis_index("s") * NC + lax.axis_index("c")
        row0 = wid * RPW
        pltpu.sync_copy(src_hbm.at[pl.ds(row0, RPW)], srcb)
        pltpu.sync_copy(dst_hbm.at[pl.ds(row0, RPW)], dstb)
        pltpu.sync_copy(a02_hbm, a02_v)
        pltpu.sync_copy(a08_hbm, a08_v)
        iota16 = lax.iota(jnp.int32, 16) * 16

        def fire(k, slot):
            for h in range(nh):
                pltpu.async_copy(xl_hbm[h].at[srcb.at[k]], gl[slot][h], sems[slot])
                pltpu.async_copy(xr_hbm[h].at[dstb.at[k]], gr[slot][h], sems[slot])

        def drain(slot):
            for h in range(nh):
                pltpu.make_async_copy(
                    xl_hbm[h].at[srcb.at[0]], gl[slot][h], sems[slot]).wait()
                pltpu.make_async_copy(
                    xr_hbm[h].at[dstb.at[0]], gr[slot][h], sems[slot]).wait()

        def compute(k, slot, mx):
            def grp(g, mx):
                for ii in range(16):
                    i = g * 16 + ii
                    acc1 = jnp.zeros((16,), jnp.float32)
                    acc2 = jnp.zeros((16,), jnp.float32)
                    for h in range(nh):
                        for j in range(DH // 32):
                            ue, uo = plsc.unpack(
                                gl[slot][h][i, pl.ds(j * 32, 32)],
                                format=plsc.PackFormat.INTERLEAVED,
                                preferred_element_type=jnp.float32)
                            we, wo = plsc.unpack(
                                gr[slot][h][i, pl.ds(j * 32, 32)],
                                format=plsc.PackFormat.INTERLEAVED,
                                preferred_element_type=jnp.float32)
                            off = (h * (DH // 32) + j) * 32
                            ze = ue + we
                            zo = uo + wo
                            acc1 = acc1 + ze * a02_v[pl.ds(off, 16)]
                            acc2 = acc2 + jnp.maximum(ze, 0.0) * a08_v[pl.ds(off, 16)]
                            acc1 = acc1 + zo * a02_v[pl.ds(off + 16, 16)]
                            acc2 = acc2 + jnp.maximum(zo, 0.0) * a08_v[pl.ds(off + 16, 16)]
                    accm[pl.ds(ii * 16, 16)] = acc1 + acc2
                rs = jnp.zeros((16,), jnp.float32)
                for j in range(16):
                    rs = rs + plsc.load_gather(accm, [iota16 + j])
                eb[k, pl.ds(g * 16, 16)] = rs
                return jnp.maximum(mx, rs)

            return lax.fori_loop(0, CH // 16, grp, mx)

        fire(0, 0)

        def pair(i, mx):
            k0 = i * 2
            fire(k0 + 1, 1)
            drain(0)
            mx = compute(k0, 0, mx)
            fire(k0 + 2, 0)
            drain(1)
            mx = compute(k0 + 1, 1, mx)
            return mx

        mx0 = jnp.full((16,), -jnp.inf, jnp.float32)
        mx = lax.fori_loop(0, (RPW - 1) // 2, pair, mx0)
        drain(0)
        mx = compute(RPW - 1, 0, mx)

        pltpu.sync_copy(eb, e_hbm.at[pl.ds(row0, RPW)])
        mx_v[...] = mx
        pltpu.sync_copy(mx_v, emax_hbm.at[wid])

    out_type = (
        jax.ShapeDtypeStruct((ECH, CH), jnp.float32),
        jax.ShapeDtypeStruct((NW, 16), jnp.float32),
    )
    scratch = (
        [pltpu.VMEM((RPW, CH), jnp.int32), pltpu.VMEM((RPW, CH), jnp.int32),
         pltpu.VMEM((RPW, CH), jnp.float32)]
        + [pltpu.VMEM((CH, DH), jnp.bfloat16) for _ in range(4 * nh)]
        + [
            pltpu.VMEM((nh * DH,), jnp.float32),
            pltpu.VMEM((nh * DH,), jnp.float32),
            pltpu.VMEM((256,), jnp.float32),
            pltpu.VMEM((16,), jnp.float32),
        ]
        + [pltpu.SemaphoreType.DMA for _ in range(2)]
    )
    return pl.kernel(
        body, out_type=out_type, mesh=_mesh(), scratch_types=scratch,
        compiler_params=_SC_PARAMS,
    )(src2, dst2, *xls, *xrs, a02, a08)


def _edge_aggregate(sd2, e2, emax, xls):
    nh = len(xls)

    def body(*refs):
        it = iter(refs)
        sd_hbm = next(it); e_hbm = next(it); emax_hbm = next(it)
        xl_hbm = [next(it) for _ in range(nh)]
        out_hbm = [next(it) for _ in range(nh)]
        sdd = [next(it), next(it)]
        eed = [next(it), next(it)]
        sda = [next(it), next(it)]
        eea = [next(it), next(it)]
        gb = [next(it), next(it)]
        rows = [next(it), next(it)]
        inv_v = next(it); buf_v = next(it); emax_v = next(it)
        denom_sh = next(it); inv_sh = next(it); acc_sh = next(it)
        lsemd = [next(it), next(it)]
        dsem = [next(it), next(it)]
        lsema = [next(it), next(it)]
        gsem = [next(it), next(it)]
        ssem = [next(it), next(it)]

        c = lax.axis_index("c")
        s = lax.axis_index("s")
        wid = s * NC + c

        pltpu.sync_copy(emax_hbm, emax_v)
        m = jnp.full((16,), -jnp.inf, jnp.float32)
        for r in range(NW):
            m = jnp.maximum(m, emax_v[r, :])
        gmax = jnp.full((16,), jnp.max(m), jnp.float32)

        for j in range(RPT // 16):
            buf_v[pl.ds(j * 16, 16)] = jnp.zeros((16,), jnp.float32)
        pltpu.sync_copy(buf_v, denom_sh.at[pl.ds(s * RPT, RPT)])
        plsc.subcore_barrier()

        drow0 = s * RPS

        def dload(k, slot):
            pltpu.async_copy(sd_hbm.at[drow0 + k], sdd[slot], lsemd[slot])
            pltpu.async_copy(e_hbm.at[drow0 + k], eed[slot], lsemd[slot])

        def dload_drain(slot):
            pltpu.make_async_copy(sd_hbm.at[drow0], sdd[slot], lsemd[slot]).wait()
            pltpu.make_async_copy(e_hbm.at[drow0], eed[slot], lsemd[slot]).wait()

        def dstep(k, slot):
            dload_drain(slot)
            for g in range(CH // 16):
                eed[slot][pl.ds(g * 16, 16)] = jnp.exp(
                    eed[slot][pl.ds(g * 16, 16)] - gmax)
            pltpu.async_copy(
                eed[slot], denom_sh.at[sdd[slot].at[1]], dsem[slot], add=True)
            pltpu.make_async_copy(
                eed[slot], denom_sh.at[sdd[slot].at[1]], dsem[slot]).wait()

        dload(0, 0)
        dload(1, 1)

        def dpair(i, carry):
            k0 = i * 2
            dstep(k0, 0)

            @pl.when(k0 + 2 < RPS)
            def _():
                dload(k0 + 2, 0)

            dstep(k0 + 1, 1)

            @pl.when(k0 + 3 < RPS)
            def _():
                dload(k0 + 3, 1)

            return carry

        lax.fori_loop(0, RPS // 2, dpair, 0)
        plsc.subcore_barrier()

        pltpu.sync_copy(denom_sh.at[pl.ds(s * RPT, RPT)], buf_v)
        for j in range(RPT // 16):
            d16 = buf_v[pl.ds(j * 16, 16)]
            buf_v[pl.ds(j * 16, 16)] = 1.0 / (d16 + 1e-16)
        pltpu.sync_copy(buf_v, inv_sh.at[pl.ds(s * RPT, RPT)])
        plsc.subcore_barrier()
        pltpu.sync_copy(inv_sh, inv_v)

        arow0 = wid * RPW

        def aload(k, slot):
            pltpu.async_copy(sd_hbm.at[arow0 + k], sda[slot], lsema[slot])
            pltpu.async_copy(e_hbm.at[arow0 + k], eea[slot], lsema[slot])

        def aload_drain(slot):
            pltpu.make_async_copy(sd_hbm.at[arow0], sda[slot], lsema[slot]).wait()
            pltpu.make_async_copy(e_hbm.at[arow0], eea[slot], lsema[slot]).wait()

        for h in range(nh):
            def zrow(r, carry):
                for j in range(DH // 16):
                    rows[0][r, pl.ds(j * 16, 16)] = jnp.zeros((16,), jnp.float32)
                return carry

            lax.fori_loop(0, CH, zrow, 0)

            def zblk(bk, carry):
                pltpu.sync_copy(rows[0], acc_sh.at[pl.ds(s * RPT + bk * CH, CH)])
                return carry

            lax.fori_loop(0, RPT // CH, zblk, 0)
            plsc.subcore_barrier()

            def gfire(slot):
                pltpu.async_copy(
                    xl_hbm[h].at[sda[slot].at[0]], gb[slot], gsem[slot])

            def gdrain(slot):
                pltpu.make_async_copy(
                    xl_hbm[h].at[sda[slot].at[0]], gb[slot], gsem[slot]).wait()

            def sfire(slot):
                pltpu.async_copy(
                    rows[slot], acc_sh.at[sda[slot].at[1]], ssem[slot], add=True)

            def sdrain(slot):
                pltpu.make_async_copy(
                    rows[slot], acc_sh.at[sda[slot].at[1]], ssem[slot]).wait()

            def alpha_scale(slot):
                def sgrp(g2, carry2):
                    invd = plsc.load_gather(
                        inv_v, [sda[slot][1, pl.ds(g2 * 16, 16)]])
                    av = jnp.exp(
                        eea[slot][pl.ds(g2 * 16, 16)] - gmax) * invd
                    for ii in range(16):
                        i = g2 * 16 + ii
                        ai = jnp.broadcast_to(av[ii], (16,))
                        for j in range(DH // 32):
                            ue, uo = plsc.unpack(
                                gb[slot][i, pl.ds(j * 32, 32)],
                                format=plsc.PackFormat.INTERLEAVED,
                                preferred_element_type=jnp.float32)
                            rows[slot][i, pl.ds(j * 32, 16)] = ue * ai
                            rows[slot][i, pl.ds(j * 32 + 16, 16)] = uo * ai
                    return carry2

                lax.fori_loop(0, CH // 16, sgrp, 0)

            aload(0, 0)
            aload_drain(0)
            gfire(0)
            aload(1, 1)

            def apair(i, carry):
                k0 = i * 2
                aload_drain(1)
                gfire(1)
                gdrain(0)
                alpha_scale(0)
                sfire(0)
                sdrain(0)
                aload(k0 + 2, 0)
                aload_drain(0)
                gfire(0)
                gdrain(1)
                alpha_scale(1)
                sfire(1)
                sdrain(1)

                @pl.when(k0 + 3 < RPW)
                def _():
                    aload(k0 + 3, 1)

                return carry

            lax.fori_loop(0, (RPW - 1) // 2, apair, 0)
            gdrain(0)
            alpha_scale(0)
            sfire(0)
            sdrain(0)
            plsc.subcore_barrier()
            pltpu.sync_copy(
                acc_sh.at[pl.ds(s * RPT, RPT)],
                out_hbm[h].at[c, pl.ds(s * RPT, RPT)],
            )
            plsc.subcore_barrier()

    out_type = tuple(
        jax.ShapeDtypeStruct((NC, NPAD, DH), jnp.float32) for _ in range(nh)
    )
    scratch = (
        [pltpu.VMEM((2, CH), jnp.int32) for _ in range(2)]
        + [pltpu.VMEM((CH,), jnp.float32) for _ in range(2)]
        + [pltpu.VMEM((2, CH), jnp.int32) for _ in range(2)]
        + [pltpu.VMEM((CH,), jnp.float32) for _ in range(2)]
        + [pltpu.VMEM((CH, DH), jnp.bfloat16) for _ in range(2)]
        + [pltpu.VMEM((CH, DH), jnp.float32) for _ in range(2)]
        + [
            pltpu.VMEM((NPAD,), jnp.float32),
            pltpu.VMEM((RPT,), jnp.float32),
            pltpu.VMEM((NW, 16), jnp.float32),
            pltpu.VMEM_SHARED((NPAD,), jnp.float32),
            pltpu.VMEM_SHARED((NPAD,), jnp.float32),
            pltpu.VMEM_SHARED((NPAD, DH), jnp.float32),
        ]
        + [pltpu.SemaphoreType.DMA for _ in range(10)]
    )
    outs = pl.kernel(
        body, out_type=out_type, mesh=_mesh(), scratch_types=scratch,
        compiler_params=_SC_PARAMS,
    )(sd2, e2, emax, *xls)
    return tuple(outs) if isinstance(outs, (list, tuple)) else (outs,)


def kernel(x, edge_index, Wl1, Wr1, a1, b1, Wres1, Wl2, Wr2, a2, b2, Wres2):
    src2 = edge_index[0].reshape(ECH, CH)
    dst2 = edge_index[1].reshape(ECH, CH)
    sd2 = jnp.stack([src2, dst2], axis=1)

    xl1, xr1, res1 = _mm3(x, Wl1, Wr1, Wres1, b1)
    xl1b, xr1b = _bfi(xl1), _bfi(xr1)
    e1, emax1 = _edge_logits(src2, dst2, [xl1b], [xr1b], LEAK * a1,
                             (1.0 - LEAK) * a1)
    (part1,) = _edge_aggregate(sd2, e1, emax1, [xl1b])

    xl2a, xl2b, xr2a, xr2b, res2a, res2b = _combine_mm(
        part1, res1, Wl2, Wr2, Wres2, b2)

    xl2ab, xl2bb = _bfi(xl2a), _bfi(xl2b)
    xr2ab, xr2bb = _bfi(xr2a), _bfi(xr2b)
    e2, emax2 = _edge_logits(src2, dst2, [xl2ab, xl2bb], [xr2ab, xr2bb],
                             LEAK * a2, (1.0 - LEAK) * a2)
    partA, partB = _edge_aggregate(sd2, e2, emax2, [xl2ab, xl2bb])

    return _final(partA, partB, res2a, res2b)


# speedup vs baseline: 1.3912x; 1.3912x over previous
import jax
import jax.numpy as jnp
from jax import lax
from jax.experimental import pallas as pl
from jax.experimental.pallas import tpu as pltpu
from jax.experimental.pallas import tpu_sc as plsc

N = 10000
E = 320000
NC = 2
NS = 16
NW = NC * NS
CH = 80
ECH = E // CH
RPW = ECH // NW
RPS = ECH // NS
NPAD = 10240
RPT = NPAD // NS
DH = 128
ROW_BLK = 1000
LEAK = 0.2


def _bfi(t):
    n, d = t.shape
    return (t.reshape(n, d // 32, 2, 16).swapaxes(-1, -2)
            .reshape(n, d).astype(jnp.bfloat16))


def _mesh():
    return plsc.VectorSubcoreMesh(
        core_axis_name="c", subcore_axis_name="s", num_cores=NC, num_subcores=NS
    )


_SC_PARAMS = pltpu.CompilerParams(
    needs_layout_passes=False, use_tc_tiling_on_sc=False)


def _mm3_body(x_ref, wl_ref, wr_ref, wres_ref, b_ref, xl_ref, xr_ref, res_ref):
    x = x_ref[...]
    xl_ref[...] = jnp.dot(x, wl_ref[...], preferred_element_type=jnp.float32)
    xr_ref[...] = jnp.dot(x, wr_ref[...], preferred_element_type=jnp.float32)
    res_ref[...] = (
        jnp.dot(x, wres_ref[...], preferred_element_type=jnp.float32) + b_ref[...]
    )


def _mm3(x, Wl, Wr, Wres, b):
    n, d_in = x.shape
    d_out = Wl.shape[1]
    out_sd = jax.ShapeDtypeStruct((n, d_out), jnp.float32)
    w_spec = pl.BlockSpec((d_in, d_out), lambda i: (0, 0))
    b_spec = pl.BlockSpec((1, d_out), lambda i: (0, 0))
    row_spec = pl.BlockSpec((ROW_BLK, d_in), lambda i: (i, 0))
    out_spec = pl.BlockSpec((ROW_BLK, d_out), lambda i: (i, 0))
    return pl.pallas_call(
        _mm3_body,
        grid=(n // ROW_BLK,),
        in_specs=[row_spec, w_spec, w_spec, w_spec, b_spec],
        out_specs=[out_spec, out_spec, out_spec],
        out_shape=[out_sd, out_sd, out_sd],
    )(x, Wl, Wr, Wres, b.reshape(1, d_out))


def _combine_mm_body(part_ref, res1_ref, *refs):
    (wla, wlb, wra, wrb, wsa, wsb, ba, bb,
     xla_ref, xlb_ref, xra_ref, xrb_ref, resa_ref, resb_ref) = refs
    h1 = jnp.maximum(part_ref[0] + part_ref[1] + res1_ref[...], 0.0)
    dot = lambda w: jnp.dot(h1, w[...], preferred_element_type=jnp.float32)
    xla_ref[...] = dot(wla)
    xlb_ref[...] = dot(wlb)
    xra_ref[...] = dot(wra)
    xrb_ref[...] = dot(wrb)
    resa_ref[...] = dot(wsa) + ba[...]
    resb_ref[...] = dot(wsb) + bb[...]


def _combine_mm(part1, res1, Wl2, Wr2, Wres2, b2):
    out_sd = jax.ShapeDtypeStruct((N, DH), jnp.float32)
    part_spec = pl.BlockSpec((NC, ROW_BLK, DH), lambda i: (0, i, 0))
    row_spec = pl.BlockSpec((ROW_BLK, DH), lambda i: (i, 0))
    w_spec = pl.BlockSpec((DH, DH), lambda i: (0, 0))
    b_spec = pl.BlockSpec((1, DH), lambda i: (0, 0))
    ws = [Wl2[:, :DH], Wl2[:, DH:], Wr2[:, :DH], Wr2[:, DH:],
          Wres2[:, :DH], Wres2[:, DH:]]
    bs = [b2[:DH].reshape(1, DH), b2[DH:].reshape(1, DH)]
    return pl.pallas_call(
        _combine_mm_body,
        grid=(N // ROW_BLK,),
        in_specs=[part_spec, row_spec] + [w_spec] * 6 + [b_spec] * 2,
        out_specs=[row_spec] * 6,
        out_shape=[out_sd] * 6,
    )(part1[:, :N], res1, *ws, *bs)


def _final_body(pa_ref, pb_ref, resa_ref, resb_ref, out_ref):
    out_ref[:, :DH] = pa_ref[0] + pa_ref[1] + resa_ref[...]
    out_ref[:, DH:] = pb_ref[0] + pb_ref[1] + resb_ref[...]


def _final(partA, partB, res2a, res2b):
    part_spec = pl.BlockSpec((NC, ROW_BLK, DH), lambda i: (0, i, 0))
    row_spec = pl.BlockSpec((ROW_BLK, DH), lambda i: (i, 0))
    return pl.pallas_call(
        _final_body,
        grid=(N // ROW_BLK,),
        in_specs=[part_spec, part_spec, row_spec, row_spec],
        out_specs=pl.BlockSpec((ROW_BLK, 2 * DH), lambda i: (i, 0)),
        out_shape=jax.ShapeDtypeStruct((N, 2 * DH), jnp.float32),
    )(partA[:, :N], partB[:, :N], res2a, res2b)


def _edge_logits(src2, dst2, xls, xrs, a02, a08):
    nh = len(xls)

    def body(*refs):
        it = iter(refs)
        src_hbm = next(it); dst_hbm = next(it)
        xl_hbm = [next(it) for _ in range(nh)]
        xr_hbm = [next(it) for _ in range(nh)]
        a02_hbm = next(it); a08_hbm = next(it)
        e_hbm = next(it); emax_hbm = next(it)
        srcb = next(it); dstb = next(it); eb = next(it)
        gl = [[next(it) for _ in range(nh)] for _ in range(2)]
        gr = [[next(it) for _ in range(nh)] for _ in range(2)]
        a02_v = next(it); a08_v = next(it); accm = next(it); mx_v = next(it)
        sems = [next(it) for _ in range(2)]

        wid = lax.axis_index("s") * NC + lax.axis_index("c")
        row0 = wid * RPW
        pltpu.sync_copy(src_hbm.at[pl.ds(row0, RPW)], srcb)
        pltpu.sync_copy(dst_hbm.at[pl.ds(row0, RPW)], dstb)
        pltpu.sync_copy(a02_hbm, a02_v)
        pltpu.sync_copy(a08_hbm, a08_v)
        iota16 = lax.iota(jnp.int32, 16) * 16

        def fire(k, slot):
            for h in range(nh):
                pltpu.async_copy(xl_hbm[h].at[srcb.at[k]], gl[slot][h], sems[slot])
                pltpu.async_copy(xr_hbm[h].at[dstb.at[k]], gr[slot][h], sems[slot])

        def drain(slot):
            for h in range(nh):
                pltpu.make_async_copy(
                    xl_hbm[h].at[srcb.at[0]], gl[slot][h], sems[slot]).wait()
                pltpu.make_async_copy(
                    xr_hbm[h].at[dstb.at[0]], gr[slot][h], sems[slot]).wait()

        def compute(k, slot, mx):
            def grp(g, mx):
                for ii in range(16):
                    i = g * 16 + ii
                    acc1 = jnp.zeros((16,), jnp.float32)
                    acc2 = jnp.zeros((16,), jnp.float32)
                    for h in range(nh):
                        for j in range(DH // 16):
                            z = (gl[slot][h][i, pl.ds(j * 16, 16)]
                                 + gr[slot][h][i, pl.ds(j * 16, 16)])
                            off = (h * (DH // 16) + j) * 16
                            acc1 = acc1 + z * a02_v[pl.ds(off, 16)]
                            acc2 = (acc2 + jnp.maximum(z, 0.0)
                                    * a08_v[pl.ds(off, 16)])
                    accm[pl.ds(ii * 16, 16)] = acc1 + acc2
                rs = jnp.zeros((16,), jnp.float32)
                for j in range(16):
                    rs = rs + plsc.load_gather(accm, [iota16 + j])
                eb[k, pl.ds(g * 16, 16)] = rs
                return jnp.maximum(mx, rs)

            return lax.fori_loop(0, CH // 16, grp, mx)

        fire(0, 0)

        def pair(i, mx):
            k0 = i * 2
            fire(k0 + 1, 1)
            drain(0)
            mx = compute(k0, 0, mx)
            fire(k0 + 2, 0)
            drain(1)
            mx = compute(k0 + 1, 1, mx)
            return mx

        mx0 = jnp.full((16,), -jnp.inf, jnp.float32)
        mx = lax.fori_loop(0, (RPW - 1) // 2, pair, mx0)
        drain(0)
        mx = compute(RPW - 1, 0, mx)

        pltpu.sync_copy(eb, e_hbm.at[pl.ds(row0, RPW)])
        mx_v[...] = mx
        pltpu.sync_copy(mx_v, emax_hbm.at[wid])

    out_type = (
        jax.ShapeDtypeStruct((ECH, CH), jnp.float32),
        jax.ShapeDtypeStruct((NW, 16), jnp.float32),
    )
    scratch = (
        [pltpu.VMEM((RPW, CH), jnp.int32), pltpu.VMEM((RPW, CH), jnp.int32),
         pltpu.VMEM((RPW, CH), jnp.float32)]
        + [pltpu.VMEM((CH, DH), jnp.float32) for _ in range(4 * nh)]
        + [
            pltpu.VMEM((nh * DH,), jnp.float32),
            pltpu.VMEM((nh * DH,), jnp.float32),
            pltpu.VMEM((256,), jnp.float32),
            pltpu.VMEM((16,), jnp.float32),
        ]
        + [pltpu.SemaphoreType.DMA for _ in range(2)]
    )
    return pl.kernel(
        body, out_type=out_type, mesh=_mesh(), scratch_types=scratch,
        compiler_params=_SC_PARAMS,
    )(src2, dst2, *xls, *xrs, a02, a08)


def _edge_aggregate(sd2, e2, emax, xls):
    nh = len(xls)

    def body(*refs):
        it = iter(refs)
        sd_hbm = next(it); e_hbm = next(it); emax_hbm = next(it)
        xl_hbm = [next(it) for _ in range(nh)]
        out_hbm = [next(it) for _ in range(nh)]
        sdd = [next(it), next(it)]
        eed = [next(it), next(it)]
        sda = [next(it), next(it)]
        eea = [next(it), next(it)]
        rows = [next(it), next(it)]
        inv_v = next(it); buf_v = next(it); emax_v = next(it)
        denom_sh = next(it); inv_sh = next(it); acc_sh = next(it)
        lsemd = [next(it), next(it)]
        dsem = [next(it), next(it)]
        lsema = [next(it), next(it)]
        gsem = [next(it), next(it)]
        ssem = [next(it), next(it)]

        c = lax.axis_index("c")
        s = lax.axis_index("s")
        wid = s * NC + c

        pltpu.sync_copy(emax_hbm, emax_v)
        m = jnp.full((16,), -jnp.inf, jnp.float32)
        for r in range(NW):
            m = jnp.maximum(m, emax_v[r, :])
        gmax = jnp.full((16,), jnp.max(m), jnp.float32)

        for j in range(RPT // 16):
            buf_v[pl.ds(j * 16, 16)] = jnp.zeros((16,), jnp.float32)
        pltpu.sync_copy(buf_v, denom_sh.at[pl.ds(s * RPT, RPT)])
        plsc.subcore_barrier()

        drow0 = s * RPS

        def dload(k, slot):
            pltpu.async_copy(sd_hbm.at[drow0 + k], sdd[slot], lsemd[slot])
            pltpu.async_copy(e_hbm.at[drow0 + k], eed[slot], lsemd[slot])

        def dload_drain(slot):
            pltpu.make_async_copy(sd_hbm.at[drow0], sdd[slot], lsemd[slot]).wait()
            pltpu.make_async_copy(e_hbm.at[drow0], eed[slot], lsemd[slot]).wait()

        def dstep(k, slot):
            dload_drain(slot)
            for g in range(CH // 16):
                eed[slot][pl.ds(g * 16, 16)] = jnp.exp(
                    eed[slot][pl.ds(g * 16, 16)] - gmax)
            pltpu.async_copy(
                eed[slot], denom_sh.at[sdd[slot].at[1]], dsem[slot], add=True)
            pltpu.make_async_copy(
                eed[slot], denom_sh.at[sdd[slot].at[1]], dsem[slot]).wait()

        dload(0, 0)
        dload(1, 1)

        def dpair(i, carry):
            k0 = i * 2
            dstep(k0, 0)

            @pl.when(k0 + 2 < RPS)
            def _():
                dload(k0 + 2, 0)

            dstep(k0 + 1, 1)

            @pl.when(k0 + 3 < RPS)
            def _():
                dload(k0 + 3, 1)

            return carry

        lax.fori_loop(0, RPS // 2, dpair, 0)
        plsc.subcore_barrier()

        pltpu.sync_copy(denom_sh.at[pl.ds(s * RPT, RPT)], buf_v)
        for j in range(RPT // 16):
            d16 = buf_v[pl.ds(j * 16, 16)]
            buf_v[pl.ds(j * 16, 16)] = 1.0 / (d16 + 1e-16)
        pltpu.sync_copy(buf_v, inv_sh.at[pl.ds(s * RPT, RPT)])
        plsc.subcore_barrier()
        pltpu.sync_copy(inv_sh, inv_v)

        arow0 = wid * RPW

        def aload(k, slot):
            pltpu.async_copy(sd_hbm.at[arow0 + k], sda[slot], lsema[slot])
            pltpu.async_copy(e_hbm.at[arow0 + k], eea[slot], lsema[slot])

        def aload_drain(slot):
            pltpu.make_async_copy(sd_hbm.at[arow0], sda[slot], lsema[slot]).wait()
            pltpu.make_async_copy(e_hbm.at[arow0], eea[slot], lsema[slot]).wait()

        for h in range(nh):
            def zrow(r, carry):
                for j in range(DH // 16):
                    rows[0][r, pl.ds(j * 16, 16)] = jnp.zeros((16,), jnp.float32)
                return carry

            lax.fori_loop(0, CH, zrow, 0)

            def zblk(bk, carry):
                pltpu.sync_copy(rows[0], acc_sh.at[pl.ds(s * RPT + bk * CH, CH)])
                return carry

            lax.fori_loop(0, RPT // CH, zblk, 0)
            plsc.subcore_barrier()

            def gfire(slot):
                pltpu.async_copy(
                    xl_hbm[h].at[sda[slot].at[0]], rows[slot], gsem[slot])

            def gdrain(slot):
                pltpu.make_async_copy(
                    xl_hbm[h].at[sda[slot].at[0]], rows[slot], gsem[slot]).wait()

            def sfire(slot):
                pltpu.async_copy(
                    rows[slot], acc_sh.at[sda[slot].at[1]], ssem[slot], add=True)

            def sdrain(slot):
                pltpu.make_async_copy(
                    rows[slot], acc_sh.at[sda[slot].at[1]], ssem[slot]).wait()

            def alpha_scale(slot):
                def sgrp(g2, carry2):
                    invd = plsc.load_gather(
                        inv_v, [sda[slot][1, pl.ds(g2 * 16, 16)]])
                    av = jnp.exp(
                        eea[slot][pl.ds(g2 * 16, 16)] - gmax) * invd
                    for ii in range(16):
                        i = g2 * 16 + ii
                        ai = jnp.broadcast_to(av[ii], (16,))
                        for j in range(DH // 16):
                            rows[slot][i, pl.ds(j * 16, 16)] = (
                                rows[slot][i, pl.ds(j * 16, 16)] * ai)
                    return carry2

                lax.fori_loop(0, CH // 16, sgrp, 0)

            aload(0, 0)
            aload_drain(0)
            gfire(0)
            aload(1, 1)

            def apair(i, carry):
                k0 = i * 2
                aload_drain(1)
                gfire(1)
                gdrain(0)
                alpha_scale(0)
                sfire(0)
                sdrain(0)
                aload(k0 + 2, 0)
                aload_drain(0)
                gfire(0)
                gdrain(1)
                alpha_scale(1)
                sfire(1)
                sdrain(1)

                @pl.when(k0 + 3 < RPW)
                def _():
                    aload(k0 + 3, 1)

                return carry

            lax.fori_loop(0, (RPW - 1) // 2, apair, 0)
            gdrain(0)
            alpha_scale(0)
            sfire(0)
            sdrain(0)
            plsc.subcore_barrier()
            pltpu.sync_copy(
                acc_sh.at[pl.ds(s * RPT, RPT)],
                out_hbm[h].at[c, pl.ds(s * RPT, RPT)],
            )
            plsc.subcore_barrier()

    out_type = tuple(
        jax.ShapeDtypeStruct((NC, NPAD, DH), jnp.float32) for _ in range(nh)
    )
    scratch = (
        [pltpu.VMEM((2, CH), jnp.int32) for _ in range(2)]
        + [pltpu.VMEM((CH,), jnp.float32) for _ in range(2)]
        + [pltpu.VMEM((2, CH), jnp.int32) for _ in range(2)]
        + [pltpu.VMEM((CH,), jnp.float32) for _ in range(2)]
        + [pltpu.VMEM((CH, DH), jnp.float32) for _ in range(2)]
        + [
            pltpu.VMEM((NPAD,), jnp.float32),
            pltpu.VMEM((RPT,), jnp.float32),
            pltpu.VMEM((NW, 16), jnp.float32),
            pltpu.VMEM_SHARED((NPAD,), jnp.float32),
            pltpu.VMEM_SHARED((NPAD,), jnp.float32),
            pltpu.VMEM_SHARED((NPAD, DH), jnp.float32),
        ]
        + [pltpu.SemaphoreType.DMA for _ in range(10)]
    )
    outs = pl.kernel(
        body, out_type=out_type, mesh=_mesh(), scratch_types=scratch,
        compiler_params=_SC_PARAMS,
    )(sd2, e2, emax, *xls)
    return tuple(outs) if isinstance(outs, (list, tuple)) else (outs,)


def kernel(x, edge_index, Wl1, Wr1, a1, b1, Wres1, Wl2, Wr2, a2, b2, Wres2):
    src2 = edge_index[0].reshape(ECH, CH)
    dst2 = edge_index[1].reshape(ECH, CH)
    sd2 = jnp.stack([src2, dst2], axis=1)

    xl1, xr1, res1 = _mm3(x, Wl1, Wr1, Wres1, b1)
    e1, emax1 = _edge_logits(src2, dst2, [xl1], [xr1], LEAK * a1,
                             (1.0 - LEAK) * a1)
    (part1,) = _edge_aggregate(sd2, e1, emax1, [xl1])

    xl2a, xl2b, xr2a, xr2b, res2a, res2b = _combine_mm(
        part1, res1, Wl2, Wr2, Wres2, b2)

    e2, emax2 = _edge_logits(src2, dst2, [xl2a, xl2b], [xr2a, xr2b],
                             LEAK * a2, (1.0 - LEAK) * a2)
    partA, partB = _edge_aggregate(sd2, e2, emax2, [xl2a, xl2b])

    return _final(partA, partB, res2a, res2b)


# speedup vs baseline: 1.5051x; 1.0818x over previous
import jax
import jax.numpy as jnp
from jax import lax
from jax.experimental import pallas as pl
from jax.experimental.pallas import tpu as pltpu
from jax.experimental.pallas import tpu_sc as plsc

N = 10000
E = 320000
NC = 2
NS = 16
NW = NC * NS
CH = 80
ECH = E // CH
RPW = ECH // NW
RPS = ECH // NS
NPAD = 10240
RPT = NPAD // NS
DH = 128
ROW_BLK = 1000
LEAK = 0.2


def _bfi(t):
    n, d = t.shape
    return (t.reshape(n, d // 32, 2, 16).swapaxes(-1, -2)
            .reshape(n, d).astype(jnp.bfloat16))


def _mesh():
    return plsc.VectorSubcoreMesh(
        core_axis_name="c", subcore_axis_name="s", num_cores=NC, num_subcores=NS
    )


_SC_PARAMS = pltpu.CompilerParams(
    needs_layout_passes=False, use_tc_tiling_on_sc=False)


def _mm3_body(x_ref, wl_ref, wr_ref, wres_ref, b_ref, xl_ref, xr_ref, res_ref):
    x = x_ref[...]
    xl_ref[...] = jnp.dot(x, wl_ref[...], preferred_element_type=jnp.float32)
    xr_ref[...] = jnp.dot(x, wr_ref[...], preferred_element_type=jnp.float32)
    res_ref[...] = (
        jnp.dot(x, wres_ref[...], preferred_element_type=jnp.float32) + b_ref[...]
    )


def _mm3(x, Wl, Wr, Wres, b):
    n, d_in = x.shape
    d_out = Wl.shape[1]
    out_sd = jax.ShapeDtypeStruct((n, d_out), jnp.float32)
    w_spec = pl.BlockSpec((d_in, d_out), lambda i: (0, 0))
    b_spec = pl.BlockSpec((1, d_out), lambda i: (0, 0))
    row_spec = pl.BlockSpec((ROW_BLK, d_in), lambda i: (i, 0))
    out_spec = pl.BlockSpec((ROW_BLK, d_out), lambda i: (i, 0))
    return pl.pallas_call(
        _mm3_body,
        grid=(n // ROW_BLK,),
        in_specs=[row_spec, w_spec, w_spec, w_spec, b_spec],
        out_specs=[out_spec, out_spec, out_spec],
        out_shape=[out_sd, out_sd, out_sd],
    )(x, Wl, Wr, Wres, b.reshape(1, d_out))


def _combine_mm_body(part_ref, res1_ref, *refs):
    (wla, wlb, wra, wrb, wsa, wsb, ba, bb,
     xla_ref, xlb_ref, xra_ref, xrb_ref, resa_ref, resb_ref) = refs
    h1 = jnp.maximum(part_ref[0] + part_ref[1] + res1_ref[...], 0.0)
    dot = lambda w: jnp.dot(h1, w[...], preferred_element_type=jnp.float32)
    xla_ref[...] = dot(wla)
    xlb_ref[...] = dot(wlb)
    xra_ref[...] = dot(wra)
    xrb_ref[...] = dot(wrb)
    resa_ref[...] = dot(wsa) + ba[...]
    resb_ref[...] = dot(wsb) + bb[...]


def _combine_mm(part1, res1, Wl2, Wr2, Wres2, b2):
    out_sd = jax.ShapeDtypeStruct((N, DH), jnp.float32)
    part_spec = pl.BlockSpec((NC, ROW_BLK, DH), lambda i: (0, i, 0))
    row_spec = pl.BlockSpec((ROW_BLK, DH), lambda i: (i, 0))
    w_spec = pl.BlockSpec((DH, DH), lambda i: (0, 0))
    b_spec = pl.BlockSpec((1, DH), lambda i: (0, 0))
    ws = [Wl2[:, :DH], Wl2[:, DH:], Wr2[:, :DH], Wr2[:, DH:],
          Wres2[:, :DH], Wres2[:, DH:]]
    bs = [b2[:DH].reshape(1, DH), b2[DH:].reshape(1, DH)]
    return pl.pallas_call(
        _combine_mm_body,
        grid=(N // ROW_BLK,),
        in_specs=[part_spec, row_spec] + [w_spec] * 6 + [b_spec] * 2,
        out_specs=[row_spec] * 6,
        out_shape=[out_sd] * 6,
    )(part1[:, :N], res1, *ws, *bs)


def _final_body(pa_ref, pb_ref, resa_ref, resb_ref, out_ref):
    out_ref[:, :DH] = pa_ref[0] + pa_ref[1] + resa_ref[...]
    out_ref[:, DH:] = pb_ref[0] + pb_ref[1] + resb_ref[...]


def _final(partA, partB, res2a, res2b):
    part_spec = pl.BlockSpec((NC, ROW_BLK, DH), lambda i: (0, i, 0))
    row_spec = pl.BlockSpec((ROW_BLK, DH), lambda i: (i, 0))
    return pl.pallas_call(
        _final_body,
        grid=(N // ROW_BLK,),
        in_specs=[part_spec, part_spec, row_spec, row_spec],
        out_specs=pl.BlockSpec((ROW_BLK, 2 * DH), lambda i: (i, 0)),
        out_shape=jax.ShapeDtypeStruct((N, 2 * DH), jnp.float32),
    )(partA[:, :N], partB[:, :N], res2a, res2b)


def _edge_logits(src2, dst2, xls, xrs, att):
    nh = len(xls)

    def body(*refs):
        it = iter(refs)
        src_hbm = next(it); dst_hbm = next(it)
        xl_hbm = [next(it) for _ in range(nh)]
        xr_hbm = [next(it) for _ in range(nh)]
        a_hbm = next(it)
        e_hbm = next(it); emax_hbm = next(it)
        srcb = next(it); dstb = next(it); eb = next(it)
        gl = [[next(it) for _ in range(nh)] for _ in range(2)]
        gr = [[next(it) for _ in range(nh)] for _ in range(2)]
        a_v = next(it); accm = next(it); mx_v = next(it)
        sems = [next(it) for _ in range(2)]

        wid = lax.axis_index("s") * NC + lax.axis_index("c")
        row0 = wid * RPW
        pltpu.sync_copy(src_hbm.at[pl.ds(row0, RPW)], srcb)
        pltpu.sync_copy(dst_hbm.at[pl.ds(row0, RPW)], dstb)
        pltpu.sync_copy(a_hbm, a_v)
        iota16 = lax.iota(jnp.int32, 16) * 16

        def fire(k, slot):
            for h in range(nh):
                pltpu.async_copy(xl_hbm[h].at[srcb.at[k]], gl[slot][h], sems[slot])
                pltpu.async_copy(xr_hbm[h].at[dstb.at[k]], gr[slot][h], sems[slot])

        def drain(slot):
            for h in range(nh):
                pltpu.make_async_copy(
                    xl_hbm[h].at[srcb.at[0]], gl[slot][h], sems[slot]).wait()
                pltpu.make_async_copy(
                    xr_hbm[h].at[dstb.at[0]], gr[slot][h], sems[slot]).wait()

        def compute(k, slot, mx):
            def grp(g, mx):
                for ii in range(16):
                    i = g * 16 + ii
                    acc = jnp.zeros((16,), jnp.float32)
                    for h in range(nh):
                        for j in range(DH // 16):
                            t = (gl[slot][h][i, pl.ds(j * 16, 16)]
                                 + gr[slot][h][i, pl.ds(j * 16, 16)])
                            t = jnp.maximum(t, LEAK * t)
                            acc = acc + t * a_v[pl.ds((h * (DH // 16) + j) * 16, 16)]
                    accm[pl.ds(ii * 16, 16)] = acc
                rs = jnp.zeros((16,), jnp.float32)
                for j in range(16):
                    rs = rs + plsc.load_gather(accm, [iota16 + j])
                eb[k, pl.ds(g * 16, 16)] = rs
                return jnp.maximum(mx, rs)

            return lax.fori_loop(0, CH // 16, grp, mx)

        fire(0, 0)

        def pair(i, mx):
            k0 = i * 2
            fire(k0 + 1, 1)
            drain(0)
            mx = compute(k0, 0, mx)
            fire(k0 + 2, 0)
            drain(1)
            mx = compute(k0 + 1, 1, mx)
            return mx

        mx0 = jnp.full((16,), -jnp.inf, jnp.float32)
        mx = lax.fori_loop(0, (RPW - 1) // 2, pair, mx0)
        drain(0)
        mx = compute(RPW - 1, 0, mx)

        pltpu.sync_copy(eb, e_hbm.at[pl.ds(row0, RPW)])
        mx_v[...] = mx
        pltpu.sync_copy(mx_v, emax_hbm.at[wid])

    out_type = (
        jax.ShapeDtypeStruct((ECH, CH), jnp.float32),
        jax.ShapeDtypeStruct((NW, 16), jnp.float32),
    )
    scratch = (
        [pltpu.VMEM((RPW, CH), jnp.int32), pltpu.VMEM((RPW, CH), jnp.int32),
         pltpu.VMEM((RPW, CH), jnp.float32)]
        + [pltpu.VMEM((CH, DH), jnp.float32) for _ in range(4 * nh)]
        + [
            pltpu.VMEM((nh * DH,), jnp.float32),
            pltpu.VMEM((256,), jnp.float32),
            pltpu.VMEM((16,), jnp.float32),
        ]
        + [pltpu.SemaphoreType.DMA for _ in range(2)]
    )
    return pl.kernel(
        body, out_type=out_type, mesh=_mesh(), scratch_types=scratch,
        compiler_params=_SC_PARAMS,
    )(src2, dst2, *xls, *xrs, att)


def _edge_aggregate(sd2, e2, emax, xls):
    nh = len(xls)

    def body(*refs):
        it = iter(refs)
        sd_hbm = next(it); e_hbm = next(it); emax_hbm = next(it)
        xl_hbm = [next(it) for _ in range(nh)]
        out_hbm = [next(it) for _ in range(nh)]
        sdd = [next(it), next(it)]
        eed = [next(it), next(it)]
        sda = [next(it), next(it)]
        eea = [next(it), next(it)]
        rows = [next(it), next(it)]
        inv_v = next(it); buf_v = next(it); emax_v = next(it)
        denom_sh = next(it); inv_sh = next(it); acc_sh = next(it)
        lsemd = [next(it), next(it)]
        dsem = [next(it), next(it)]
        lsema = [next(it), next(it)]
        gsem = [next(it), next(it)]
        ssem = [next(it), next(it)]

        c = lax.axis_index("c")
        s = lax.axis_index("s")
        wid = s * NC + c

        pltpu.sync_copy(emax_hbm, emax_v)
        m = jnp.full((16,), -jnp.inf, jnp.float32)
        for r in range(NW):
            m = jnp.maximum(m, emax_v[r, :])
        gmax = jnp.full((16,), jnp.max(m), jnp.float32)

        for j in range(RPT // 16):
            buf_v[pl.ds(j * 16, 16)] = jnp.zeros((16,), jnp.float32)
        pltpu.sync_copy(buf_v, denom_sh.at[pl.ds(s * RPT, RPT)])
        plsc.subcore_barrier()

        drow0 = s * RPS

        def dload(k, slot):
            pltpu.async_copy(sd_hbm.at[drow0 + k], sdd[slot], lsemd[slot])
            pltpu.async_copy(e_hbm.at[drow0 + k], eed[slot], lsemd[slot])

        def dload_drain(slot):
            pltpu.make_async_copy(sd_hbm.at[drow0], sdd[slot], lsemd[slot]).wait()
            pltpu.make_async_copy(e_hbm.at[drow0], eed[slot], lsemd[slot]).wait()

        def dstep(k, slot):
            dload_drain(slot)
            for g in range(CH // 16):
                eed[slot][pl.ds(g * 16, 16)] = jnp.exp(
                    eed[slot][pl.ds(g * 16, 16)] - gmax)
            pltpu.async_copy(
                eed[slot], denom_sh.at[sdd[slot].at[1]], dsem[slot], add=True)
            pltpu.make_async_copy(
                eed[slot], denom_sh.at[sdd[slot].at[1]], dsem[slot]).wait()

        dload(0, 0)
        dload(1, 1)

        def dpair(i, carry):
            k0 = i * 2
            dstep(k0, 0)

            @pl.when(k0 + 2 < RPS)
            def _():
                dload(k0 + 2, 0)

            dstep(k0 + 1, 1)

            @pl.when(k0 + 3 < RPS)
            def _():
                dload(k0 + 3, 1)

            return carry

        lax.fori_loop(0, RPS // 2, dpair, 0)
        plsc.subcore_barrier()

        pltpu.sync_copy(denom_sh.at[pl.ds(s * RPT, RPT)], buf_v)
        for j in range(RPT // 16):
            d16 = buf_v[pl.ds(j * 16, 16)]
            buf_v[pl.ds(j * 16, 16)] = 1.0 / (d16 + 1e-16)
        pltpu.sync_copy(buf_v, inv_sh.at[pl.ds(s * RPT, RPT)])
        plsc.subcore_barrier()
        pltpu.sync_copy(inv_sh, inv_v)

        arow0 = wid * RPW

        def aload(k, slot):
            pltpu.async_copy(sd_hbm.at[arow0 + k], sda[slot], lsema[slot])
            pltpu.async_copy(e_hbm.at[arow0 + k], eea[slot], lsema[slot])

        def aload_drain(slot):
            pltpu.make_async_copy(sd_hbm.at[arow0], sda[slot], lsema[slot]).wait()
            pltpu.make_async_copy(e_hbm.at[arow0], eea[slot], lsema[slot]).wait()

        for h in range(nh):
            def zrow(r, carry):
                for j in range(DH // 16):
                    rows[0][r, pl.ds(j * 16, 16)] = jnp.zeros((16,), jnp.float32)
                return carry

            lax.fori_loop(0, CH, zrow, 0)

            def zblk(bk, carry):
                pltpu.sync_copy(rows[0], acc_sh.at[pl.ds(s * RPT + bk * CH, CH)])
                return carry

            lax.fori_loop(0, RPT // CH, zblk, 0)
            plsc.subcore_barrier()

            def gfire(slot):
                pltpu.async_copy(
                    xl_hbm[h].at[sda[slot].at[0]], rows[slot], gsem[slot])

            def gdrain(slot):
                pltpu.make_async_copy(
                    xl_hbm[h].at[sda[slot].at[0]], rows[slot], gsem[slot]).wait()

            def sfire(slot):
                pltpu.async_copy(
                    rows[slot], acc_sh.at[sda[slot].at[1]], ssem[slot], add=True)

            def sdrain(slot):
                pltpu.make_async_copy(
                    rows[slot], acc_sh.at[sda[slot].at[1]], ssem[slot]).wait()

            def alpha_scale(slot):
                def sgrp(g2, carry2):
                    invd = plsc.load_gather(
                        inv_v, [sda[slot][1, pl.ds(g2 * 16, 16)]])
                    av = jnp.exp(
                        eea[slot][pl.ds(g2 * 16, 16)] - gmax) * invd
                    for ii in range(16):
                        i = g2 * 16 + ii
                        ai = jnp.broadcast_to(av[ii], (16,))
                        for j in range(DH // 16):
                            rows[slot][i, pl.ds(j * 16, 16)] = (
                                rows[slot][i, pl.ds(j * 16, 16)] * ai)
                    return carry2

                lax.fori_loop(0, CH // 16, sgrp, 0)

            aload(0, 0)
            aload_drain(0)
            gfire(0)
            aload(1, 1)

            def apair(i, carry):
                k0 = i * 2
                aload_drain(1)
                gfire(1)
                gdrain(0)
                alpha_scale(0)
                sfire(0)
                sdrain(0)
                aload(k0 + 2, 0)
                aload_drain(0)
                gfire(0)
                gdrain(1)
                alpha_scale(1)
                sfire(1)
                sdrain(1)

                @pl.when(k0 + 3 < RPW)
                def _():
                    aload(k0 + 3, 1)

                return carry

            lax.fori_loop(0, (RPW - 1) // 2, apair, 0)
            gdrain(0)
            alpha_scale(0)
            sfire(0)
            sdrain(0)
            plsc.subcore_barrier()
            pltpu.sync_copy(
                acc_sh.at[pl.ds(s * RPT, RPT)],
                out_hbm[h].at[c, pl.ds(s * RPT, RPT)],
            )
            plsc.subcore_barrier()

    out_type = tuple(
        jax.ShapeDtypeStruct((NC, NPAD, DH), jnp.float32) for _ in range(nh)
    )
    scratch = (
        [pltpu.VMEM((2, CH), jnp.int32) for _ in range(2)]
        + [pltpu.VMEM((CH,), jnp.float32) for _ in range(2)]
        + [pltpu.VMEM((2, CH), jnp.int32) for _ in range(2)]
        + [pltpu.VMEM((CH,), jnp.float32) for _ in range(2)]
        + [pltpu.VMEM((CH, DH), jnp.float32) for _ in range(2)]
        + [
            pltpu.VMEM((NPAD,), jnp.float32),
            pltpu.VMEM((RPT,), jnp.float32),
            pltpu.VMEM((NW, 16), jnp.float32),
            pltpu.VMEM_SHARED((NPAD,), jnp.float32),
            pltpu.VMEM_SHARED((NPAD,), jnp.float32),
            pltpu.VMEM_SHARED((NPAD, DH), jnp.float32),
        ]
        + [pltpu.SemaphoreType.DMA for _ in range(10)]
    )
    outs = pl.kernel(
        body, out_type=out_type, mesh=_mesh(), scratch_types=scratch,
        compiler_params=_SC_PARAMS,
    )(sd2, e2, emax, *xls)
    return tuple(outs) if isinstance(outs, (list, tuple)) else (outs,)


def kernel(x, edge_index, Wl1, Wr1, a1, b1, Wres1, Wl2, Wr2, a2, b2, Wres2):
    src2 = edge_index[0].reshape(ECH, CH)
    dst2 = edge_index[1].reshape(ECH, CH)
    sd2 = jnp.stack([src2, dst2], axis=1)

    xl1, xr1, res1 = _mm3(x, Wl1, Wr1, Wres1, b1)
    e1, emax1 = _edge_logits(src2, dst2, [xl1], [xr1], a1)
    (part1,) = _edge_aggregate(sd2, e1, emax1, [xl1])

    xl2a, xl2b, xr2a, xr2b, res2a, res2b = _combine_mm(
        part1, res1, Wl2, Wr2, Wres2, b2)

    e2, emax2 = _edge_logits(src2, dst2, [xl2a, xl2b], [xr2a, xr2b], a2)
    partA, partB = _edge_aggregate(sd2, e2, emax2, [xl2a, xl2b])

    return _final(partA, partB, res2a, res2b)
